# baseline (speedup 1.0000x reference)
import functools

import jax
import jax.numpy as jnp
from jax import lax
from jax.experimental import pallas as pl
from jax.experimental.pallas import tpu as pltpu
from jax.experimental.pallas import tpu_sc as plsc

N = 10000
N_PAD = 10240
D = 128
H1 = 256
H2 = 128
E = 320000
E_PAD = 327680
E_LBL = 100000
E_LBL_PAD = 102400

NC = 2
NS = 16
CH = 128
GS = 16
RPT = N_PAD // NS
CPT = E_PAD // NS // CH
CPT2 = E_PAD // (NC * NS) // CH
CPT_F = 128
CPT_S = 32
LPT = E_LBL_PAD // NS // CH

_MESH = plsc.VectorSubcoreMesh(core_axis_name="c", subcore_axis_name="s")


def _f32(shape):
    return jax.ShapeDtypeStruct(shape, jnp.float32)


@functools.partial(
    pl.kernel,
    out_type=[_f32((N_PAD, 16)), _f32((N_PAD, 16))],
    mesh=_MESH,
    scratch_types=[
        pltpu.VMEM((CPT, 1, CH), jnp.int32),
        pltpu.VMEM((CH, 16), jnp.float32),
        pltpu.VMEM_SHARED((N_PAD, 16), jnp.float32),
        pltpu.SemaphoreType.DMA,
    ],
)
def _sc_degree(dst_gg, dst_rev, ones_hbm, deg_gg, deg_rev, idx_all, ones_v, acc_sh, ssem):
    cid = lax.axis_index("c")
    sid = lax.axis_index("s")
    r0 = sid * RPT
    pltpu.sync_copy(ones_hbm.at[pl.ds(r0, RPT)], acc_sh.at[pl.ds(r0, RPT)])
    pltpu.sync_copy(ones_hbm.at[pl.ds(0, CH)], ones_v)

    def run(dst_hbm):
        pltpu.sync_copy(dst_hbm.at[pl.ds(sid * CPT, CPT)], idx_all)
        plsc.subcore_barrier()
        nb = 8

        def body(g, carry):
            ds = [pltpu.async_copy(ones_v, acc_sh.at[idx_all.at[g * nb + b, 0]],
                                   ssem, add=True) for b in range(nb)]
            for d in ds:
                d.wait()
            return carry
        lax.fori_loop(0, CPT // nb, body, 0)

    @pl.when(cid == 0)
    def _():
        run(dst_gg)

    @pl.when(cid == 1)
    def _():
        run(dst_rev)

    plsc.subcore_barrier()

    @pl.when(cid == 0)
    def _():
        pltpu.sync_copy(acc_sh.at[pl.ds(r0, RPT)], deg_gg.at[pl.ds(r0, RPT)])

    @pl.when(cid == 1)
    def _():
        pltpu.sync_copy(acc_sh.at[pl.ds(r0, RPT)], deg_rev.at[pl.ds(r0, RPT)])


def _pipelined_scatter(y_hbm, acc_sh, src_hbm, dst_hbm, c0, sidx, didx, rows,
                       isem, gsem, ssem, n_chunks):
    ng = n_chunks // GS
    pltpu.sync_copy(src_hbm.at[pl.ds(c0, GS)], sidx.at[pl.ds(0, GS)])
    pltpu.sync_copy(dst_hbm.at[pl.ds(c0, GS)], didx.at[pl.ds(0, GS)])

    def outer(g, carry):
        p = lax.rem(g, 2)
        pb = p * GS
        q = (1 - p) * GS
        o = c0 + (g + 1) * GS

        @pl.when(g + 1 < ng)
        def _():
            pltpu.async_copy(src_hbm.at[pl.ds(o, GS)], sidx.at[pl.ds(q, GS)], isem)
            pltpu.async_copy(dst_hbm.at[pl.ds(o, GS)], didx.at[pl.ds(q, GS)], isem)

        gd = pltpu.async_copy(y_hbm.at[sidx.at[pb, 0]], rows[0], gsem)
        sd = [None, None]
        for k in range(GS):
            b = k % 2
            gd.wait()
            if k + 1 < GS:
                if sd[1 - b] is not None:
                    sd[1 - b].wait()
                gd = pltpu.async_copy(y_hbm.at[sidx.at[pb + k + 1, 0]], rows[1 - b], gsem)
            sd[b] = pltpu.async_copy(rows[b], acc_sh.at[didx.at[pb + k, 0]],
                                     ssem, add=True)
        sd[0].wait()
        sd[1].wait()

        @pl.when(g + 1 < ng)
        def _():
            pltpu.make_async_copy(src_hbm.at[pl.ds(o, GS)], sidx.at[pl.ds(q, GS)], isem).wait()
            pltpu.make_async_copy(dst_hbm.at[pl.ds(o, GS)], didx.at[pl.ds(q, GS)], isem).wait()
        return carry

    lax.fori_loop(0, ng, outer, 0)


@functools.partial(
    pl.kernel,
    out_type=[_f32((N_PAD, D))] * 4,
    mesh=_MESH,
    scratch_types=[
        pltpu.VMEM((2 * GS, 1, CH), jnp.int32),
        pltpu.VMEM((2 * GS, 1, CH), jnp.int32),
        [pltpu.VMEM((CH, D), jnp.float32)] * 2,
        pltpu.VMEM_SHARED((N_PAD, D), jnp.float32),
        pltpu.SemaphoreType.DMA,
        pltpu.SemaphoreType.DMA,
        pltpu.SemaphoreType.DMA,
    ],
)
def _sc_scatter(y_gg0, y_gg1, y_rev0, y_rev1, src_gg, dst_gg, src_rev, dst_rev,
                p_gg_0, p_gg_1, p_rev_0, p_rev_1,
                sidx, didx, rows, acc_sh, isem, gsem, ssem):
    cid = lax.axis_index("c")
    sid = lax.axis_index("s")
    r0 = sid * RPT

    def run(y_hbm, init_hbm, src_hbm, dst_hbm, out_hbm, c0, n_chunks):
        pltpu.sync_copy(init_hbm.at[pl.ds(r0, RPT)], acc_sh.at[pl.ds(r0, RPT)])
        plsc.subcore_barrier()
        _pipelined_scatter(y_hbm, acc_sh, src_hbm, dst_hbm, c0,
                           sidx, didx, rows, isem, gsem, ssem, n_chunks)
        plsc.subcore_barrier()
        pltpu.sync_copy(acc_sh.at[pl.ds(r0, RPT)], out_hbm.at[pl.ds(r0, RPT)])
        plsc.subcore_barrier()

    @pl.when(cid == 0)
    def _():
        run(y_gg0, y_gg0, src_gg, dst_gg, p_gg_0, sid * CPT2, CPT2)
        run(y_rev0, y_rev0, src_rev, dst_rev, p_rev_0, sid * CPT2, CPT2)

    @pl.when(cid == 1)
    def _():
        run(y_gg1, y_gg1, src_gg, dst_gg, p_gg_1, NS * CPT2 + sid * CPT2, CPT2)
        run(y_rev1, y_rev1, src_rev, dst_rev, p_rev_1, NS * CPT2 + sid * CPT2, CPT2)


@functools.partial(
    pl.kernel,
    out_type=[_f32((E_LBL_PAD, D)), _f32((E_LBL_PAD, D))],
    mesh=_MESH,
    scratch_types=[
        pltpu.VMEM((LPT, 1, CH), jnp.int32),
        [pltpu.VMEM((CH, D), jnp.float32)] * 5,
        pltpu.SemaphoreType.DMA,
        pltpu.SemaphoreType.DMA,
    ],
)
def _sc_gather_lbl(g2, i0, i1, ef1, ef2, idx_all, rows, gsem, wsem):
    cid = lax.axis_index("c")
    sid = lax.axis_index("s")
    nb = 5

    def run(idx_hbm, out_hbm):
        pltpu.sync_copy(idx_hbm.at[pl.ds(sid * LPT, LPT)], idx_all)

        def body(g, carry):
            gd = [pltpu.async_copy(g2.at[idx_all.at[g * nb + b, 0]], rows[b], gsem)
                  for b in range(nb)]
            wd = []
            for b in range(nb):
                j = g * nb + b
                gd[b].wait()
                wd.append(pltpu.async_copy(
                    rows[b], out_hbm.at[pl.ds((sid * LPT + j) * CH, CH)], wsem))
            for d in wd:
                d.wait()
            return carry
        lax.fori_loop(0, LPT // nb, body, 0)

    @pl.when(cid == 0)
    def _():
        run(i0, ef1)

    @pl.when(cid == 1)
    def _():
        run(i1, ef2)


BM = 1024


def _dis(deg_block):
    return lax.rsqrt(deg_block[:, :1])


def _tc_prescale_body(x_ref, dgg_ref, drev_ref,
                      xt_gg0_ref, xt_gg1_ref, xt_rev0_ref, xt_rev1_ref):
    x = x_ref[...]
    xt_gg = x * _dis(dgg_ref[...])
    xt_rev = x * _dis(drev_ref[...])
    xt_gg0_ref[...] = xt_gg
    xt_gg1_ref[...] = xt_gg
    xt_rev0_ref[...] = xt_rev
    xt_rev1_ref[...] = xt_rev


def _tc_prescale(x, deg_gg, deg_rev):
    row = lambda i: (i, 0)
    return pl.pallas_call(
        _tc_prescale_body,
        grid=(N_PAD // BM,),
        in_specs=[
            pl.BlockSpec((BM, D), row),
            pl.BlockSpec((BM, 16), row),
            pl.BlockSpec((BM, 16), row),
        ],
        out_specs=[pl.BlockSpec((BM, D), row)] * 4,
        out_shape=[_f32((N_PAD, D))] * 4,
    )(x, deg_gg, deg_rev)


def _tc_mid_body(pgg0_ref, pgg1_ref, prev0_ref, prev1_ref,
                 xtgg_ref, xtrev_ref,
                 dgg_ref, drev_ref, b1gg_ref, b1rev_ref,
                 w1gg_ref, w1rev_ref, w2gg_ref, w2rev_ref,
                 y2gg0_ref, y2gg1_ref, y2rev0_ref, y2rev1_ref):
    dis_gg = _dis(dgg_ref[...])
    dis_rev = _dis(drev_ref[...])
    agg_gg = pgg0_ref[...] + pgg1_ref[...] - xtgg_ref[...]
    agg_rev = prev0_ref[...] + prev1_ref[...] - xtrev_ref[...]
    g = jnp.dot(agg_gg, w1gg_ref[...], preferred_element_type=jnp.float32) * dis_gg \
        + b1gg_ref[...] \
        + jnp.dot(agg_rev, w1rev_ref[...], preferred_element_type=jnp.float32) * dis_rev \
        + b1rev_ref[...]
    g = jnp.maximum(g, 0.0)
    y2gg = jnp.dot(g, w2gg_ref[...], preferred_element_type=jnp.float32) * dis_gg
    y2rev = jnp.dot(g, w2rev_ref[...], preferred_element_type=jnp.float32) * dis_rev
    y2gg0_ref[...] = y2gg
    y2gg1_ref[...] = y2gg
    y2rev0_ref[...] = y2rev
    y2rev1_ref[...] = y2rev


def _tc_mid(acc, xt_gg, xt_rev, deg_gg, deg_rev, b1gg, b1rev, w1gg, w1rev, w2gg, w2rev):
    row = lambda i: (i, 0)
    full = lambda i: (0, 0)
    return pl.pallas_call(
        _tc_mid_body,
        grid=(N_PAD // BM,),
        in_specs=[
            pl.BlockSpec((BM, D), row)] * 6 + [
            pl.BlockSpec((BM, 16), row),
            pl.BlockSpec((BM, 16), row),
            pl.BlockSpec((1, H1), full),
            pl.BlockSpec((1, H1), full),
            pl.BlockSpec((D, H1), full),
            pl.BlockSpec((D, H1), full),
            pl.BlockSpec((H1, H2), full),
            pl.BlockSpec((H1, H2), full),
        ],
        out_specs=[pl.BlockSpec((BM, H2), row)] * 4,
        out_shape=[_f32((N_PAD, H2))] * 4,
    )(*acc, xt_gg, xt_rev, deg_gg, deg_rev, b1gg, b1rev, w1gg, w1rev, w2gg, w2rev)


def _tc_g2_body(pgg0_ref, pgg1_ref, prev0_ref, prev1_ref,
                ygg_ref, yrev_ref,
                dgg_ref, drev_ref, b2gg_ref, b2rev_ref, g2_ref):
    dis_gg = _dis(dgg_ref[...])
    dis_rev = _dis(drev_ref[...])
    gg = pgg0_ref[...] + pgg1_ref[...] - ygg_ref[...]
    rv = prev0_ref[...] + prev1_ref[...] - yrev_ref[...]
    g2_ref[...] = gg * dis_gg + b2gg_ref[...] + rv * dis_rev + b2rev_ref[...]


def _tc_g2(acc, y2gg, y2rev, deg_gg, deg_rev, b2gg, b2rev):
    row = lambda i: (i, 0)
    full = lambda i: (0, 0)
    return pl.pallas_call(
        _tc_g2_body,
        grid=(N_PAD // BM,),
        in_specs=[
            pl.BlockSpec((BM, H2), row)] * 6 + [
            pl.BlockSpec((BM, 16), row),
            pl.BlockSpec((BM, 16), row),
            pl.BlockSpec((1, H2), full),
            pl.BlockSpec((1, H2), full),
        ],
        out_specs=pl.BlockSpec((BM, H2), row),
        out_shape=_f32((N_PAD, H2)),
    )(*acc, y2gg, y2rev, deg_gg, deg_rev, b2gg, b2rev)


def _tc_dot_body(a_ref, b_ref, o_ref):
    o_ref[...] = jnp.sum(a_ref[...] * b_ref[...], axis=1, keepdims=True)


def _tc_dot(ef1, ef2):
    bm = 1024
    row = lambda i: (i, 0)
    return pl.pallas_call(
        _tc_dot_body,
        grid=(E_LBL_PAD // bm,),
        in_specs=[pl.BlockSpec((bm, D), row)] * 2,
        out_specs=pl.BlockSpec((bm, 1), row),
        out_shape=_f32((E_LBL_PAD, 1)),
    )(ef1, ef2)


def kernel(x_gene, x_cell, W1_gg, b1_gg, W1_rev, b1_rev, W1_cc, b1_cc,
           W2_gg, b2_gg, W2_rev, b2_rev, W2_cc, b2_cc,
           edge_index_gg, edge_index_gg_rev, edge_index_cc, edge_label_index):
    epad = jnp.full((E_PAD - E,), N, jnp.int32)
    dpad = N + jnp.arange(E_PAD - E, dtype=jnp.int32) % (N_PAD - N)
    src_gg = jnp.concatenate([edge_index_gg[0], epad]).reshape(-1, 1, CH)
    dst_gg = jnp.concatenate([edge_index_gg[1], dpad]).reshape(-1, 1, CH)
    src_rev = jnp.concatenate([edge_index_gg_rev[0], epad]).reshape(-1, 1, CH)
    dst_rev = jnp.concatenate([edge_index_gg_rev[1], dpad]).reshape(-1, 1, CH)
    pad = jnp.zeros((E_LBL_PAD - E_LBL,), jnp.int32)
    i0 = jnp.concatenate([edge_label_index[0], pad]).reshape(-1, 1, CH)
    i1 = jnp.concatenate([edge_label_index[1], pad]).reshape(-1, 1, CH)
    ones16 = jnp.ones((N_PAD, 16), jnp.float32)
    xg = jnp.pad(x_gene, ((0, N_PAD - N), (0, 0)))

    deg_gg, deg_rev = _sc_degree(dst_gg, dst_rev, ones16)

    xt = _tc_prescale(xg, deg_gg, deg_rev)
    acc1 = _sc_scatter(*xt, src_gg, dst_gg, src_rev, dst_rev)
    y2 = _tc_mid(acc1, xt[0], xt[2], deg_gg, deg_rev,
                 b1_gg.reshape(1, H1), b1_rev.reshape(1, H1),
                 W1_gg, W1_rev, W2_gg, W2_rev)
    acc2 = _sc_scatter(*y2, src_gg, dst_gg, src_rev, dst_rev)
    g2 = _tc_g2(acc2, y2[0], y2[2], deg_gg, deg_rev,
                b2_gg.reshape(1, H2), b2_rev.reshape(1, H2))
    ef1, ef2 = _sc_gather_lbl(g2, i0, i1)
    pred = _tc_dot(ef1, ef2)
    return pred[:E_LBL, 0]


# speedup vs baseline: 2.8184x; 2.8184x over previous
import functools

import jax
import jax.numpy as jnp
from jax import lax
from jax.experimental import pallas as pl
from jax.experimental.pallas import tpu as pltpu
from jax.experimental.pallas import tpu_sc as plsc

N = 10000
N_PAD = 10240
D = 128
H1 = 256
H2 = 128
E = 320000
E_PAD = 327680
E_LBL = 100000
E_LBL_PAD = 102400

NC = 2
NS = 16
CH = 128
GS = 16
RPT = N_PAD // NS
CPT = E_PAD // NS // CH
CPT2 = E_PAD // (NC * NS) // CH
CPT_F = 128
CPT_S = 32
LPT = E_LBL_PAD // NS // CH

_MESH = plsc.VectorSubcoreMesh(core_axis_name="c", subcore_axis_name="s")


def _f32(shape):
    return jax.ShapeDtypeStruct(shape, jnp.float32)


@functools.partial(
    pl.kernel,
    out_type=[_f32((N_PAD, 16)), _f32((N_PAD, 16))],
    mesh=_MESH,
    scratch_types=[
        pltpu.VMEM((CPT, 1, CH), jnp.int32),
        pltpu.VMEM((CH, 16), jnp.float32),
        pltpu.VMEM_SHARED((N_PAD, 16), jnp.float32),
        pltpu.SemaphoreType.DMA,
    ],
)
def _sc_degree(dst_gg, dst_rev, ones_hbm, deg_gg, deg_rev, idx_all, ones_v, acc_sh, ssem):
    cid = lax.axis_index("c")
    sid = lax.axis_index("s")
    r0 = sid * RPT
    pltpu.sync_copy(ones_hbm.at[pl.ds(r0, RPT)], acc_sh.at[pl.ds(r0, RPT)])
    pltpu.sync_copy(ones_hbm.at[pl.ds(0, CH)], ones_v)

    def run(dst_hbm):
        pltpu.sync_copy(dst_hbm.at[pl.ds(sid * CPT, CPT)], idx_all)
        plsc.subcore_barrier()
        nb = 8

        def body(g, carry):
            ds = [pltpu.async_copy(ones_v, acc_sh.at[idx_all.at[g * nb + b, 0]],
                                   ssem, add=True) for b in range(nb)]
            for d in ds:
                d.wait()
            return carry
        lax.fori_loop(0, CPT // nb, body, 0)

    @pl.when(cid == 0)
    def _():
        run(dst_gg)

    @pl.when(cid == 1)
    def _():
        run(dst_rev)

    plsc.subcore_barrier()

    @pl.when(cid == 0)
    def _():
        pltpu.sync_copy(acc_sh.at[pl.ds(r0, RPT)], deg_gg.at[pl.ds(r0, RPT)])

    @pl.when(cid == 1)
    def _():
        pltpu.sync_copy(acc_sh.at[pl.ds(r0, RPT)], deg_rev.at[pl.ds(r0, RPT)])


def _pipelined_scatter(y_hbm, acc_sh, src_hbm, dst_hbm, c0, sidx, didx, rows,
                       isem, gsem, ssem, n_chunks):
    ng = n_chunks // GS
    pltpu.sync_copy(src_hbm.at[pl.ds(c0, GS)], sidx.at[pl.ds(0, GS)])
    pltpu.sync_copy(dst_hbm.at[pl.ds(c0, GS)], didx.at[pl.ds(0, GS)])

    def outer(g, carry):
        p = lax.rem(g, 2)
        pb = p * GS
        q = (1 - p) * GS
        o = c0 + (g + 1) * GS

        @pl.when(g + 1 < ng)
        def _():
            pltpu.async_copy(src_hbm.at[pl.ds(o, GS)], sidx.at[pl.ds(q, GS)], isem)
            pltpu.async_copy(dst_hbm.at[pl.ds(o, GS)], didx.at[pl.ds(q, GS)], isem)

        gd = pltpu.async_copy(y_hbm.at[sidx.at[pb, 0]], rows[0], gsem)
        sd = [None, None]
        for k in range(GS):
            b = k % 2
            gd.wait()
            if k + 1 < GS:
                if sd[1 - b] is not None:
                    sd[1 - b].wait()
                gd = pltpu.async_copy(y_hbm.at[sidx.at[pb + k + 1, 0]], rows[1 - b], gsem)
            sd[b] = pltpu.async_copy(rows[b], acc_sh.at[didx.at[pb + k, 0]],
                                     ssem, add=True)
        sd[0].wait()
        sd[1].wait()

        @pl.when(g + 1 < ng)
        def _():
            pltpu.make_async_copy(src_hbm.at[pl.ds(o, GS)], sidx.at[pl.ds(q, GS)], isem).wait()
            pltpu.make_async_copy(dst_hbm.at[pl.ds(o, GS)], didx.at[pl.ds(q, GS)], isem).wait()
        return carry

    lax.fori_loop(0, ng, outer, 0)


@functools.partial(
    pl.kernel,
    out_type=[_f32((N_PAD, D))] * 4,
    mesh=_MESH,
    scratch_types=[
        pltpu.VMEM((2 * GS, 1, CH), jnp.int32),
        pltpu.VMEM((2 * GS, 1, CH), jnp.int32),
        [pltpu.VMEM((CH, D), jnp.float32)] * 2,
        pltpu.VMEM_SHARED((N_PAD, D), jnp.float32),
        pltpu.SemaphoreType.DMA,
        pltpu.SemaphoreType.DMA,
        pltpu.SemaphoreType.DMA,
    ],
)
def _sc_scatter(y_gg0, y_gg1, y_rev0, y_rev1, src_gg, dst_gg, src_rev, dst_rev,
                p_gg_0, p_gg_1, p_rev_0, p_rev_1,
                sidx, didx, rows, acc_sh, isem, gsem, ssem):
    cid = lax.axis_index("c")
    sid = lax.axis_index("s")
    r0 = sid * RPT

    def run(y_hbm, init_hbm, src_hbm, dst_hbm, out_hbm, c0, n_chunks):
        pltpu.sync_copy(init_hbm.at[pl.ds(r0, RPT)], acc_sh.at[pl.ds(r0, RPT)])
        plsc.subcore_barrier()
        _pipelined_scatter(y_hbm, acc_sh, src_hbm, dst_hbm, c0,
                           sidx, didx, rows, isem, gsem, ssem, n_chunks)
        plsc.subcore_barrier()
        pltpu.sync_copy(acc_sh.at[pl.ds(r0, RPT)], out_hbm.at[pl.ds(r0, RPT)])
        plsc.subcore_barrier()

    @pl.when(cid == 0)
    def _():
        run(y_gg0, y_gg0, src_gg, dst_gg, p_gg_0, sid * CPT2, CPT2)
        run(y_rev0, y_rev0, src_rev, dst_rev, p_rev_0, sid * CPT2, CPT2)

    @pl.when(cid == 1)
    def _():
        run(y_gg1, y_gg1, src_gg, dst_gg, p_gg_1, NS * CPT2 + sid * CPT2, CPT2)
        run(y_rev1, y_rev1, src_rev, dst_rev, p_rev_1, NS * CPT2 + sid * CPT2, CPT2)


@functools.partial(
    pl.kernel,
    out_type=[_f32((E_LBL_PAD, D)), _f32((E_LBL_PAD, D))],
    mesh=_MESH,
    scratch_types=[
        pltpu.VMEM((LPT, 1, CH), jnp.int32),
        [pltpu.VMEM((CH, D), jnp.float32)] * 5,
        pltpu.SemaphoreType.DMA,
        pltpu.SemaphoreType.DMA,
    ],
)
def _sc_gather_lbl(g2, i0, i1, ef1, ef2, idx_all, rows, gsem, wsem):
    cid = lax.axis_index("c")
    sid = lax.axis_index("s")
    nb = 5

    def run(idx_hbm, out_hbm):
        pltpu.sync_copy(idx_hbm.at[pl.ds(sid * LPT, LPT)], idx_all)

        def body(g, carry):
            gd = [pltpu.async_copy(g2.at[idx_all.at[g * nb + b, 0]], rows[b], gsem)
                  for b in range(nb)]
            wd = []
            for b in range(nb):
                j = g * nb + b
                gd[b].wait()
                wd.append(pltpu.async_copy(
                    rows[b], out_hbm.at[pl.ds((sid * LPT + j) * CH, CH)], wsem))
            for d in wd:
                d.wait()
            return carry
        lax.fori_loop(0, LPT // nb, body, 0)

    @pl.when(cid == 0)
    def _():
        run(i0, ef1)

    @pl.when(cid == 1)
    def _():
        run(i1, ef2)


BM = 1024


def _dis(deg_block):
    return lax.rsqrt(deg_block[:, :1])


def _tc_prescale_body(x_ref, dgg_ref, drev_ref,
                      xt_gg0_ref, xt_gg1_ref, xt_rev0_ref, xt_rev1_ref):
    x = x_ref[...]
    xt_gg = x * _dis(dgg_ref[...])
    xt_rev = x * _dis(drev_ref[...])
    xt_gg0_ref[...] = xt_gg
    xt_gg1_ref[...] = xt_gg
    xt_rev0_ref[...] = xt_rev
    xt_rev1_ref[...] = xt_rev


def _tc_prescale(x, deg_gg, deg_rev):
    row = lambda i: (i, 0)
    return pl.pallas_call(
        _tc_prescale_body,
        grid=(N_PAD // BM,),
        in_specs=[
            pl.BlockSpec((BM, D), row),
            pl.BlockSpec((BM, 16), row),
            pl.BlockSpec((BM, 16), row),
        ],
        out_specs=[pl.BlockSpec((BM, D), row)] * 4,
        out_shape=[_f32((N_PAD, D))] * 4,
    )(x, deg_gg, deg_rev)


def _tc_mid_body(pgg0_ref, pgg1_ref, prev0_ref, prev1_ref,
                 xtgg_ref, xtrev_ref,
                 dgg_ref, drev_ref, b1gg_ref, b1rev_ref,
                 w1gg_ref, w1rev_ref, w2gg_ref, w2rev_ref,
                 y2gg0_ref, y2gg1_ref, y2rev0_ref, y2rev1_ref):
    dis_gg = _dis(dgg_ref[...])
    dis_rev = _dis(drev_ref[...])
    agg_gg = pgg0_ref[...] + pgg1_ref[...] - xtgg_ref[...]
    agg_rev = prev0_ref[...] + prev1_ref[...] - xtrev_ref[...]
    g = jnp.dot(agg_gg, w1gg_ref[...], preferred_element_type=jnp.float32) * dis_gg \
        + b1gg_ref[...] \
        + jnp.dot(agg_rev, w1rev_ref[...], preferred_element_type=jnp.float32) * dis_rev \
        + b1rev_ref[...]
    g = jnp.maximum(g, 0.0)
    y2gg = jnp.dot(g, w2gg_ref[...], preferred_element_type=jnp.float32) * dis_gg
    y2rev = jnp.dot(g, w2rev_ref[...], preferred_element_type=jnp.float32) * dis_rev
    y2gg0_ref[...] = y2gg
    y2gg1_ref[...] = y2gg
    y2rev0_ref[...] = y2rev
    y2rev1_ref[...] = y2rev


def _tc_mid(acc, xt_gg, xt_rev, deg_gg, deg_rev, b1gg, b1rev, w1gg, w1rev, w2gg, w2rev):
    row = lambda i: (i, 0)
    full = lambda i: (0, 0)
    return pl.pallas_call(
        _tc_mid_body,
        grid=(N_PAD // BM,),
        in_specs=[
            pl.BlockSpec((BM, D), row)] * 6 + [
            pl.BlockSpec((BM, 16), row),
            pl.BlockSpec((BM, 16), row),
            pl.BlockSpec((1, H1), full),
            pl.BlockSpec((1, H1), full),
            pl.BlockSpec((D, H1), full),
            pl.BlockSpec((D, H1), full),
            pl.BlockSpec((H1, H2), full),
            pl.BlockSpec((H1, H2), full),
        ],
        out_specs=[pl.BlockSpec((BM, H2), row)] * 4,
        out_shape=[_f32((N_PAD, H2))] * 4,
    )(*acc, xt_gg, xt_rev, deg_gg, deg_rev, b1gg, b1rev, w1gg, w1rev, w2gg, w2rev)


def _tc_g2_body(pgg0_ref, pgg1_ref, prev0_ref, prev1_ref,
                ygg_ref, yrev_ref,
                dgg_ref, drev_ref, b2gg_ref, b2rev_ref, g2_ref):
    dis_gg = _dis(dgg_ref[...])
    dis_rev = _dis(drev_ref[...])
    gg = pgg0_ref[...] + pgg1_ref[...] - ygg_ref[...]
    rv = prev0_ref[...] + prev1_ref[...] - yrev_ref[...]
    g2_ref[...] = gg * dis_gg + b2gg_ref[...] + rv * dis_rev + b2rev_ref[...]


def _tc_g2(acc, y2gg, y2rev, deg_gg, deg_rev, b2gg, b2rev):
    row = lambda i: (i, 0)
    full = lambda i: (0, 0)
    return pl.pallas_call(
        _tc_g2_body,
        grid=(N_PAD // BM,),
        in_specs=[
            pl.BlockSpec((BM, H2), row)] * 6 + [
            pl.BlockSpec((BM, 16), row),
            pl.BlockSpec((BM, 16), row),
            pl.BlockSpec((1, H2), full),
            pl.BlockSpec((1, H2), full),
        ],
        out_specs=pl.BlockSpec((BM, H2), row),
        out_shape=_f32((N_PAD, H2)),
    )(*acc, y2gg, y2rev, deg_gg, deg_rev, b2gg, b2rev)


def _tc_dot_body(a_ref, b_ref, o_ref):
    o_ref[...] = jnp.sum(a_ref[...] * b_ref[...], axis=1, keepdims=True)


def _tc_dot(ef1, ef2):
    bm = 1024
    row = lambda i: (i, 0)
    return pl.pallas_call(
        _tc_dot_body,
        grid=(E_LBL_PAD // bm,),
        in_specs=[pl.BlockSpec((bm, D), row)] * 2,
        out_specs=pl.BlockSpec((bm, 1), row),
        out_shape=_f32((E_LBL_PAD, 1)),
    )(ef1, ef2)


def kernel(x_gene, x_cell, W1_gg, b1_gg, W1_rev, b1_rev, W1_cc, b1_cc,
           W2_gg, b2_gg, W2_rev, b2_rev, W2_cc, b2_cc,
           edge_index_gg, edge_index_gg_rev, edge_index_cc, edge_label_index):
    dpad = N + jnp.arange(E_PAD - E, dtype=jnp.int32) % (N_PAD - N)
    src_gg = jnp.concatenate([edge_index_gg[0], dpad]).reshape(-1, 1, CH)
    dst_gg = jnp.concatenate([edge_index_gg[1], dpad]).reshape(-1, 1, CH)
    src_rev = jnp.concatenate([edge_index_gg_rev[0], dpad]).reshape(-1, 1, CH)
    dst_rev = jnp.concatenate([edge_index_gg_rev[1], dpad]).reshape(-1, 1, CH)
    lpad = jnp.arange(E_LBL_PAD - E_LBL, dtype=jnp.int32) % N
    i0 = jnp.concatenate([edge_label_index[0], lpad]).reshape(-1, 1, CH)
    i1 = jnp.concatenate([edge_label_index[1], lpad]).reshape(-1, 1, CH)
    ones16 = jnp.ones((N_PAD, 16), jnp.float32)
    xg = jnp.pad(x_gene, ((0, N_PAD - N), (0, 0)))

    deg_gg, deg_rev = _sc_degree(dst_gg, dst_rev, ones16)

    xt = _tc_prescale(xg, deg_gg, deg_rev)
    acc1 = _sc_scatter(*xt, src_gg, dst_gg, src_rev, dst_rev)
    y2 = _tc_mid(acc1, xt[0], xt[2], deg_gg, deg_rev,
                 b1_gg.reshape(1, H1), b1_rev.reshape(1, H1),
                 W1_gg, W1_rev, W2_gg, W2_rev)
    acc2 = _sc_scatter(*y2, src_gg, dst_gg, src_rev, dst_rev)
    g2 = _tc_g2(acc2, y2[0], y2[2], deg_gg, deg_rev,
                b2_gg.reshape(1, H2), b2_rev.reshape(1, H2))
    ef1, ef2 = _sc_gather_lbl(g2, i0, i1)
    pred = _tc_dot(ef1, ef2)
    return pred[:E_LBL, 0]


# speedup vs baseline: 2.9851x; 1.0592x over previous
import functools

import jax
import jax.numpy as jnp
from jax import lax
from jax.experimental import pallas as pl
from jax.experimental.pallas import tpu as pltpu
from jax.experimental.pallas import tpu_sc as plsc

N = 10000
N_PAD = 10240
D = 128
H1 = 256
H2 = 128
E = 320000
E_PAD = 327680
E_LBL = 100000
E_LBL_PAD = 102400

NC = 2
NS = 16
CH = 128
GS = 16
RPT = N_PAD // NS
CPT = E_PAD // NS // CH
CPT2 = E_PAD // (NC * NS) // CH
CPT_F = 128
CPT_S = 32
LPT = E_LBL_PAD // NS // CH

_MESH = plsc.VectorSubcoreMesh(core_axis_name="c", subcore_axis_name="s")


def _f32(shape):
    return jax.ShapeDtypeStruct(shape, jnp.float32)


@functools.partial(
    pl.kernel,
    out_type=[_f32((N_PAD, 16)), _f32((N_PAD, 16))],
    mesh=_MESH,
    scratch_types=[
        pltpu.VMEM((CPT, 1, CH), jnp.int32),
        pltpu.VMEM((CH, 16), jnp.float32),
        pltpu.VMEM_SHARED((N_PAD, 16), jnp.float32),
        pltpu.SemaphoreType.DMA,
    ],
)
def _sc_degree(dst_gg, dst_rev, ones_hbm, deg_gg, deg_rev, idx_all, ones_v, acc_sh, ssem):
    cid = lax.axis_index("c")
    sid = lax.axis_index("s")
    r0 = sid * RPT
    pltpu.sync_copy(ones_hbm.at[pl.ds(r0, RPT)], acc_sh.at[pl.ds(r0, RPT)])
    pltpu.sync_copy(ones_hbm.at[pl.ds(0, CH)], ones_v)

    def run(dst_hbm):
        pltpu.sync_copy(dst_hbm.at[pl.ds(sid * CPT, CPT)], idx_all)
        plsc.subcore_barrier()
        nb = 8

        def body(g, carry):
            ds = [pltpu.async_copy(ones_v, acc_sh.at[idx_all.at[g * nb + b, 0]],
                                   ssem, add=True) for b in range(nb)]
            for d in ds:
                d.wait()
            return carry
        lax.fori_loop(0, CPT // nb, body, 0)

    @pl.when(cid == 0)
    def _():
        run(dst_gg)

    @pl.when(cid == 1)
    def _():
        run(dst_rev)

    plsc.subcore_barrier()

    @pl.when(cid == 0)
    def _():
        pltpu.sync_copy(acc_sh.at[pl.ds(r0, RPT)], deg_gg.at[pl.ds(r0, RPT)])

    @pl.when(cid == 1)
    def _():
        pltpu.sync_copy(acc_sh.at[pl.ds(r0, RPT)], deg_rev.at[pl.ds(r0, RPT)])


def _pipelined_scatter(y_hbm, acc_sh, src_hbm, dst_hbm, c0, sidx, didx, rows,
                       isem, gsem, ssem, n_chunks):
    ng = n_chunks // GS
    pltpu.sync_copy(src_hbm.at[pl.ds(c0, GS)], sidx.at[pl.ds(0, GS)])
    pltpu.sync_copy(dst_hbm.at[pl.ds(c0, GS)], didx.at[pl.ds(0, GS)])

    def outer(g, carry):
        p = lax.rem(g, 2)
        pb = p * GS
        q = (1 - p) * GS
        o = c0 + (g + 1) * GS

        @pl.when(g + 1 < ng)
        def _():
            pltpu.async_copy(src_hbm.at[pl.ds(o, GS)], sidx.at[pl.ds(q, GS)], isem)
            pltpu.async_copy(dst_hbm.at[pl.ds(o, GS)], didx.at[pl.ds(q, GS)], isem)

        gd = pltpu.async_copy(y_hbm.at[sidx.at[pb, 0]], rows[0], gsem)
        sd = [None, None]
        for k in range(GS):
            b = k % 2
            gd.wait()
            if k + 1 < GS:
                if sd[1 - b] is not None:
                    sd[1 - b].wait()
                gd = pltpu.async_copy(y_hbm.at[sidx.at[pb + k + 1, 0]], rows[1 - b], gsem)
            sd[b] = pltpu.async_copy(rows[b], acc_sh.at[didx.at[pb + k, 0]],
                                     ssem, add=True)
        sd[0].wait()
        sd[1].wait()

        @pl.when(g + 1 < ng)
        def _():
            pltpu.make_async_copy(src_hbm.at[pl.ds(o, GS)], sidx.at[pl.ds(q, GS)], isem).wait()
            pltpu.make_async_copy(dst_hbm.at[pl.ds(o, GS)], didx.at[pl.ds(q, GS)], isem).wait()
        return carry

    lax.fori_loop(0, ng, outer, 0)


@functools.partial(
    pl.kernel,
    out_type=[_f32((N_PAD, D))] * 2,
    mesh=_MESH,
    scratch_types=[
        pltpu.VMEM((2 * GS, 1, CH), jnp.int32),
        pltpu.VMEM((2 * GS, 1, CH), jnp.int32),
        [pltpu.VMEM((CH, D), jnp.float32)] * 2,
        pltpu.VMEM_SHARED((N_PAD, D), jnp.float32),
        pltpu.SemaphoreType.DMA,
        pltpu.SemaphoreType.DMA,
        pltpu.SemaphoreType.DMA,
    ],
)
def _sc_scatter(y_gg, y_rev, src_gg, dst_gg, src_rev, dst_rev,
                acc_gg, acc_rev,
                sidx, didx, rows, acc_sh, isem, gsem, ssem):
    cid = lax.axis_index("c")
    sid = lax.axis_index("s")
    r0 = sid * RPT

    def run(y_hbm, src_hbm, dst_hbm, out_hbm):
        pltpu.sync_copy(y_hbm.at[pl.ds(r0, RPT)], acc_sh.at[pl.ds(r0, RPT)])
        plsc.subcore_barrier()
        _pipelined_scatter(y_hbm, acc_sh, src_hbm, dst_hbm, sid * CPT,
                           sidx, didx, rows, isem, gsem, ssem, CPT)
        plsc.subcore_barrier()
        pltpu.sync_copy(acc_sh.at[pl.ds(r0, RPT)], out_hbm.at[pl.ds(r0, RPT)])

    @pl.when(cid == 0)
    def _():
        run(y_gg, src_gg, dst_gg, acc_gg)

    @pl.when(cid == 1)
    def _():
        run(y_rev, src_rev, dst_rev, acc_rev)


@functools.partial(
    pl.kernel,
    out_type=[_f32((E_LBL_PAD, D)), _f32((E_LBL_PAD, D))],
    mesh=_MESH,
    scratch_types=[
        pltpu.VMEM((LPT, 1, CH), jnp.int32),
        [pltpu.VMEM((CH, D), jnp.float32)] * 5,
        pltpu.SemaphoreType.DMA,
        pltpu.SemaphoreType.DMA,
    ],
)
def _sc_gather_lbl(g2, i0, i1, ef1, ef2, idx_all, rows, gsem, wsem):
    cid = lax.axis_index("c")
    sid = lax.axis_index("s")
    nb = 5

    def run(idx_hbm, out_hbm):
        pltpu.sync_copy(idx_hbm.at[pl.ds(sid * LPT, LPT)], idx_all)

        def body(g, carry):
            gd = [pltpu.async_copy(g2.at[idx_all.at[g * nb + b, 0]], rows[b], gsem)
                  for b in range(nb)]
            wd = []
            for b in range(nb):
                j = g * nb + b
                gd[b].wait()
                wd.append(pltpu.async_copy(
                    rows[b], out_hbm.at[pl.ds((sid * LPT + j) * CH, CH)], wsem))
            for d in wd:
                d.wait()
            return carry
        lax.fori_loop(0, LPT // nb, body, 0)

    @pl.when(cid == 0)
    def _():
        run(i0, ef1)

    @pl.when(cid == 1)
    def _():
        run(i1, ef2)


BM = 1024


def _dis(deg_block):
    return lax.rsqrt(deg_block[:, :1])


def _tc_prescale_body(x_ref, dgg_ref, drev_ref, xt_gg_ref, xt_rev_ref):
    x = x_ref[...]
    xt_gg_ref[...] = x * _dis(dgg_ref[...])
    xt_rev_ref[...] = x * _dis(drev_ref[...])


def _tc_prescale(x, deg_gg, deg_rev):
    row = lambda i: (i, 0)
    return pl.pallas_call(
        _tc_prescale_body,
        grid=(N_PAD // BM,),
        in_specs=[
            pl.BlockSpec((BM, D), row),
            pl.BlockSpec((BM, 16), row),
            pl.BlockSpec((BM, 16), row),
        ],
        out_specs=[pl.BlockSpec((BM, D), row)] * 2,
        out_shape=[_f32((N_PAD, D))] * 2,
    )(x, deg_gg, deg_rev)


def _tc_mid_body(agg_gg_ref, agg_rev_ref,
                 dgg_ref, drev_ref, b1gg_ref, b1rev_ref,
                 w1gg_ref, w1rev_ref, w2gg_ref, w2rev_ref,
                 y2gg_ref, y2rev_ref):
    dis_gg = _dis(dgg_ref[...])
    dis_rev = _dis(drev_ref[...])
    agg_gg = agg_gg_ref[...]
    agg_rev = agg_rev_ref[...]
    g = jnp.dot(agg_gg, w1gg_ref[...], preferred_element_type=jnp.float32) * dis_gg \
        + b1gg_ref[...] \
        + jnp.dot(agg_rev, w1rev_ref[...], preferred_element_type=jnp.float32) * dis_rev \
        + b1rev_ref[...]
    g = jnp.maximum(g, 0.0)
    y2gg_ref[...] = jnp.dot(g, w2gg_ref[...], preferred_element_type=jnp.float32) * dis_gg
    y2rev_ref[...] = jnp.dot(g, w2rev_ref[...], preferred_element_type=jnp.float32) * dis_rev


def _tc_mid(acc, deg_gg, deg_rev, b1gg, b1rev, w1gg, w1rev, w2gg, w2rev):
    row = lambda i: (i, 0)
    full = lambda i: (0, 0)
    return pl.pallas_call(
        _tc_mid_body,
        grid=(N_PAD // BM,),
        in_specs=[
            pl.BlockSpec((BM, D), row)] * 2 + [
            pl.BlockSpec((BM, 16), row),
            pl.BlockSpec((BM, 16), row),
            pl.BlockSpec((1, H1), full),
            pl.BlockSpec((1, H1), full),
            pl.BlockSpec((D, H1), full),
            pl.BlockSpec((D, H1), full),
            pl.BlockSpec((H1, H2), full),
            pl.BlockSpec((H1, H2), full),
        ],
        out_specs=[pl.BlockSpec((BM, H2), row)] * 2,
        out_shape=[_f32((N_PAD, H2))] * 2,
    )(*acc, deg_gg, deg_rev, b1gg, b1rev, w1gg, w1rev, w2gg, w2rev)


def _tc_g2_body(agg_ref, arev_ref,
                dgg_ref, drev_ref, b2gg_ref, b2rev_ref, g2_ref):
    dis_gg = _dis(dgg_ref[...])
    dis_rev = _dis(drev_ref[...])
    gg = agg_ref[...]
    rv = arev_ref[...]
    g2_ref[...] = gg * dis_gg + b2gg_ref[...] + rv * dis_rev + b2rev_ref[...]


def _tc_g2(acc, deg_gg, deg_rev, b2gg, b2rev):
    row = lambda i: (i, 0)
    full = lambda i: (0, 0)
    return pl.pallas_call(
        _tc_g2_body,
        grid=(N_PAD // BM,),
        in_specs=[
            pl.BlockSpec((BM, H2), row)] * 2 + [
            pl.BlockSpec((BM, 16), row),
            pl.BlockSpec((BM, 16), row),
            pl.BlockSpec((1, H2), full),
            pl.BlockSpec((1, H2), full),
        ],
        out_specs=pl.BlockSpec((BM, H2), row),
        out_shape=_f32((N_PAD, H2)),
    )(*acc, deg_gg, deg_rev, b2gg, b2rev)


def _tc_dot_body(a_ref, b_ref, o_ref):
    o_ref[...] = jnp.sum(a_ref[...] * b_ref[...], axis=1, keepdims=True)


def _tc_dot(ef1, ef2):
    bm = 1024
    row = lambda i: (i, 0)
    return pl.pallas_call(
        _tc_dot_body,
        grid=(E_LBL_PAD // bm,),
        in_specs=[pl.BlockSpec((bm, D), row)] * 2,
        out_specs=pl.BlockSpec((bm, 1), row),
        out_shape=_f32((E_LBL_PAD, 1)),
    )(ef1, ef2)


def kernel(x_gene, x_cell, W1_gg, b1_gg, W1_rev, b1_rev, W1_cc, b1_cc,
           W2_gg, b2_gg, W2_rev, b2_rev, W2_cc, b2_cc,
           edge_index_gg, edge_index_gg_rev, edge_index_cc, edge_label_index):
    dpad = N + jnp.arange(E_PAD - E, dtype=jnp.int32) % (N_PAD - N)
    src_gg = jnp.concatenate([edge_index_gg[0], dpad]).reshape(-1, 1, CH)
    dst_gg = jnp.concatenate([edge_index_gg[1], dpad]).reshape(-1, 1, CH)
    src_rev = jnp.concatenate([edge_index_gg_rev[0], dpad]).reshape(-1, 1, CH)
    dst_rev = jnp.concatenate([edge_index_gg_rev[1], dpad]).reshape(-1, 1, CH)
    lpad = jnp.arange(E_LBL_PAD - E_LBL, dtype=jnp.int32) % N
    i0 = jnp.concatenate([edge_label_index[0], lpad]).reshape(-1, 1, CH)
    i1 = jnp.concatenate([edge_label_index[1], lpad]).reshape(-1, 1, CH)
    ones16 = jnp.ones((N_PAD, 16), jnp.float32)
    xg = jnp.pad(x_gene, ((0, N_PAD - N), (0, 0)))

    deg_gg, deg_rev = _sc_degree(dst_gg, dst_rev, ones16)

    xt = _tc_prescale(xg, deg_gg, deg_rev)
    acc1 = _sc_scatter(*xt, src_gg, dst_gg, src_rev, dst_rev)
    y2 = _tc_mid(acc1, deg_gg, deg_rev,
                 b1_gg.reshape(1, H1), b1_rev.reshape(1, H1),
                 W1_gg, W1_rev, W2_gg, W2_rev)
    acc2 = _sc_scatter(*y2, src_gg, dst_gg, src_rev, dst_rev)
    g2 = _tc_g2(acc2, deg_gg, deg_rev,
                b2_gg.reshape(1, H2), b2_rev.reshape(1, H2))
    ef1, ef2 = _sc_gather_lbl(g2, i0, i1)
    pred = _tc_dot(ef1, ef2)
    return pred[:E_LBL, 0]
